# baseline (speedup 1.0000x reference)
import jax
import jax.numpy as jnp
from jax import lax
from jax.experimental import pallas as pl
from jax.experimental.pallas import tpu as pltpu
from jax.experimental.pallas import tpu_sc as plsc

NSEG = 512
DIM = 256
NROWS = 160000
NC, NS, LANES = 2, 16, 16
NW = NC * NS

TCBLK = 800
NTCB = 128
SEGW = 64
NSEGP = NSEG + SEGW
TCROWS = TCBLK * NTCB
WROWS = (NROWS - TCROWS) // NW
CHUNK = 160
NCH = (WROWS + CHUNK - 1) // CHUNK
NCHP = NCH + (NCH % 2)
NT = DIM // LANES
SIDE_W = DIM + 2 * LANES
CNTW = 16
ZROWS = NSEG // NS


def _sc_segmean(feat, idx1, sum_core, cnt_core, side,
                fbuf_a, fbuf_b, ibuf_a, ibuf_b, stg, stgc, zb, zcb,
                sem_a, sem_b):
    cid = lax.axis_index("c")
    sid = lax.axis_index("s")
    wid = sid * NC + cid

    zeros = jnp.zeros((LANES,), jnp.float32)
    f32 = jnp.float32

    def splat_f(x):
        return zeros + x.astype(f32)

    def zero_zrow(r, carry):
        for j in range(NT):
            zb[r, pl.ds(j * LANES, LANES)] = zeros
        zcb[r, pl.ds(0, LANES)] = zeros
        return carry
    lax.fori_loop(0, ZROWS, zero_zrow, None)
    for j in range(SIDE_W // LANES):
        stg[0, pl.ds(j * LANES, LANES)] = zeros
    stgc[0, pl.ds(0, LANES)] = zeros

    rows16 = pl.ds(sid * ZROWS, ZROWS)
    pltpu.sync_copy(zb, sum_core.at[cid].at[rows16])
    pltpu.sync_copy(zcb, cnt_core.at[cid].at[rows16])
    plsc.subcore_barrier()

    r0 = TCROWS + wid * WROWS

    def fill_stg(cur, n, acc):
        for t in range(NT):
            stg[0, pl.ds(t * LANES, LANES)] = acc[t]
        stg[0, pl.ds(DIM, LANES)] = splat_f(n)
        stg[0, pl.ds(DIM + LANES, LANES)] = splat_f(cur)

    def flush(cur, n, ff, acc):
        fill_stg(cur, n, acc)

        @pl.when(ff == 0)
        def _():
            pltpu.sync_copy(stg, side.at[pl.ds(2 * wid, 1)])

        @pl.when(ff != 0)
        def _():
            pltpu.sync_copy(stg.at[:, pl.ds(0, DIM)],
                            sum_core.at[cid].at[pl.ds(cur, 1)])
            stgc[0, pl.ds(0, LANES)] = splat_f(n)
            pltpu.sync_copy(stgc, cnt_core.at[cid].at[pl.ds(cur, 1)])

    def chunk_base(k):
        gbase = r0 + k * CHUNK
        return gbase, jnp.minimum(gbase, NROWS - CHUNK)

    def start_dma(k, fb, ib, sem):
        _, cbase = chunk_base(k)
        pltpu.async_copy(feat.at[pl.ds(cbase, CHUNK)], fb, sem)
        pltpu.async_copy(idx1.at[pl.ds(cbase, CHUNK)], ib.at[pl.ds(0, CHUNK)],
                         sem)

    def wait_dma(k, fb, ib, sem):
        _, cbase = chunk_base(k)
        pltpu.make_async_copy(feat.at[pl.ds(cbase, CHUNK)], fb, sem).wait()
        pltpu.make_async_copy(idx1.at[pl.ds(cbase, CHUNK)],
                              ib.at[pl.ds(0, CHUNK)], sem).wait()

    izeros = jnp.zeros((LANES,), jnp.int32)

    def process(k, fbuf, ibuf, st):
        gbase, cbase = chunk_base(k)
        off = gbase - cbase
        hi = off + jnp.minimum(CHUNK, WROWS - k * CHUNK)

        def iread(j):
            return jnp.max(plsc.load_gather(ibuf, [izeros + j]))

        def row_add(j, acc):
            return tuple(acc[t] + fbuf[j, pl.ds(t * LANES, LANES)]
                         for t in range(NT))

        def row_slow(j, st):
            cur, n, ff = st[0], st[1], st[2]
            acc = st[3:]
            s = iread(j)
            changed = s != cur

            @pl.when(jnp.logical_and(changed, cur >= 0))
            def _():
                flush(cur, n, ff, acc)

            newseg = jnp.logical_and(changed, cur >= 0)
            ff = jnp.where(newseg, 1, ff)
            n = jnp.where(changed, 0, n) + 1
            acc = tuple(
                jnp.where(changed, zeros, acc[t])
                + fbuf[j, pl.ds(t * LANES, LANES)]
                for t in range(NT)
            )
            return (s, n, ff) + acc

        zero32 = jnp.int32(0)

        def group_step(g, st):
            cur, n, ff = st[0], st[1], st[2]
            j0 = off + LANES * g
            ghi = jnp.minimum(j0 + LANES, hi)
            iv = ibuf[pl.ds(j0, LANES)]
            gfast = jnp.all(iv == izeros + cur)
            acc = lax.fori_loop(jnp.where(gfast, j0, zero32),
                                jnp.where(gfast, ghi, zero32),
                                row_add, st[3:])
            n = n + jnp.where(gfast, ghi - j0, 0)
            return lax.fori_loop(jnp.where(gfast, zero32, j0),
                                 jnp.where(gfast, zero32, ghi),
                                 row_slow, (cur, n, ff) + acc)

        ngroups = (hi - off + LANES - 1) // LANES
        return lax.fori_loop(0, ngroups, group_step, st)

    init = (jnp.int32(-1), jnp.int32(0), jnp.int32(0)) + tuple(
        zeros for _ in range(NT))

    start_dma(jnp.int32(0), fbuf_a, ibuf_a, sem_a)

    def pair_step(k2, st):
        k = 2 * k2
        wait_dma(k, fbuf_a, ibuf_a, sem_a)
        start_dma(k + 1, fbuf_b, ibuf_b, sem_b)
        st = process(k, fbuf_a, ibuf_a, st)
        wait_dma(k + 1, fbuf_b, ibuf_b, sem_b)

        @pl.when(k + 2 < NCHP)
        def _():
            start_dma(k + 2, fbuf_a, ibuf_a, sem_a)
        return process(k + 1, fbuf_b, ibuf_b, st)

    st = lax.fori_loop(0, NCHP // 2, pair_step, init)
    cur, n, ff = st[0], st[1], st[2]
    acc = st[3:]

    fill_stg(cur, n, acc)
    pltpu.sync_copy(stg, side.at[pl.ds(2 * wid + 1, 1)])

    @pl.when(ff == 0)
    def _():
        stg[0, pl.ds(DIM, LANES)] = zeros
        stg[0, pl.ds(DIM + LANES, LANES)] = splat_f(jnp.int32(-1))
        pltpu.sync_copy(stg, side.at[pl.ds(2 * wid, 1)])


def _tc_partials(idx_ref, feat_ref, sum_ref, cnt_ref):
    i = pl.program_id(0)
    ids = idx_ref[0]
    fb = feat_ref[...].astype(jnp.bfloat16)
    ones_c = jnp.ones((TCBLK, 1), jnp.bfloat16)

    @pl.when(i == 0)
    def _():
        sum_ref[...] = jnp.zeros((NSEGP, DIM), jnp.float32)
        cnt_ref[...] = jnp.zeros((NSEGP, 1), jnp.float32)

    first = jnp.min(ids)
    last = jnp.max(ids)
    base8 = (first // 8) * 8
    segw = base8 + lax.broadcasted_iota(jnp.int32, (SEGW, TCBLK), 0)
    oh = jnp.where(ids == segw, 1.0, 0.0).astype(jnp.bfloat16)
    part = jnp.dot(oh, fb, preferred_element_type=jnp.float32)
    cnt = jnp.dot(oh, ones_c, preferred_element_type=jnp.float32)
    rows = pl.ds(base8, SEGW)
    sum_ref[rows, :] += part
    cnt_ref[rows, :] += cnt

    @pl.when(last - base8 >= SEGW)
    def _():
        segf = lax.broadcasted_iota(jnp.int32, (NSEG, TCBLK), 0)
        outside = jnp.logical_or(segf < base8, segf >= base8 + SEGW)
        ohf = jnp.where(jnp.logical_and(ids == segf, outside),
                        1.0, 0.0).astype(jnp.bfloat16)
        allrows = pl.ds(0, NSEG)
        sum_ref[allrows, :] += jnp.dot(ohf, fb,
                                       preferred_element_type=jnp.float32)
        cnt_ref[allrows, :] += jnp.dot(ohf, ones_c,
                                       preferred_element_type=jnp.float32)


def _combine_body(sum_ref, cnt_ref, side_ref, tsum_ref, tcnt_ref, out_ref):
    sums = sum_ref[0] + sum_ref[1] + tsum_ref[pl.ds(0, NSEG), :]
    cnts = (cnt_ref[0][:, 0:1] + cnt_ref[1][:, 0:1]
            + tcnt_ref[pl.ds(0, NSEG), :])
    side = side_ref[...]
    ids = side[:, DIM + LANES:DIM + LANES + 1]
    seg_iota = lax.broadcasted_iota(jnp.int32, (NW * 2, NSEG), 1).astype(
        jnp.float32)
    onehot = (ids == seg_iota).astype(jnp.float32)
    side_c = lax.dot_general(onehot, side,
                             (((0,), (0,)), ((), ())),
                             preferred_element_type=jnp.float32)
    total = sums + side_c[:, :DIM]
    cnt = cnts + side_c[:, DIM:DIM + 1]
    out_ref[...] = total / jnp.maximum(cnt, 1.0)


def kernel(node_features, batch_index):
    idx1 = batch_index.astype(jnp.int32)

    sc = pl.kernel(
        _sc_segmean,
        out_type=[
            jax.ShapeDtypeStruct((NC, NSEG, DIM), jnp.float32),
            jax.ShapeDtypeStruct((NC, NSEG, CNTW), jnp.float32),
            jax.ShapeDtypeStruct((NW * 2, SIDE_W), jnp.float32),
        ],
        mesh=plsc.VectorSubcoreMesh(
            core_axis_name="c", subcore_axis_name="s",
            num_cores=NC, num_subcores=NS,
        ),
        compiler_params=pltpu.CompilerParams(needs_layout_passes=False),
        scratch_types=[
            pltpu.VMEM((CHUNK, DIM), jnp.float32),
            pltpu.VMEM((CHUNK, DIM), jnp.float32),
            pltpu.VMEM((CHUNK + LANES,), jnp.int32),
            pltpu.VMEM((CHUNK + LANES,), jnp.int32),
            pltpu.VMEM((1, SIDE_W), jnp.float32),
            pltpu.VMEM((1, CNTW), jnp.float32),
            pltpu.VMEM((ZROWS, DIM), jnp.float32),
            pltpu.VMEM((ZROWS, CNTW), jnp.float32),
            pltpu.SemaphoreType.DMA,
            pltpu.SemaphoreType.DMA,
        ],
    )
    sum_core, cnt_core, side = sc(node_features, idx1)

    idx3 = idx1.reshape(NROWS // TCBLK, 1, TCBLK)
    tsum, tcnt = pl.pallas_call(
        _tc_partials,
        grid=(NTCB,),
        in_specs=[
            pl.BlockSpec((1, 1, TCBLK), lambda i: (i, 0, 0)),
            pl.BlockSpec((TCBLK, DIM), lambda i: (i, 0)),
        ],
        out_specs=[
            pl.BlockSpec((NSEGP, DIM), lambda i: (0, 0)),
            pl.BlockSpec((NSEGP, 1), lambda i: (0, 0)),
        ],
        out_shape=[
            jax.ShapeDtypeStruct((NSEGP, DIM), jnp.float32),
            jax.ShapeDtypeStruct((NSEGP, 1), jnp.float32),
        ],
    )(idx3, node_features)

    out = pl.pallas_call(
        _combine_body,
        out_shape=jax.ShapeDtypeStruct((NSEG, DIM), jnp.float32),
    )(sum_core, cnt_core, side, tsum, tcnt)
    return out


# speedup vs baseline: 1.5218x; 1.5218x over previous
import jax
import jax.numpy as jnp
from jax import lax
from jax.experimental import pallas as pl
from jax.experimental.pallas import tpu as pltpu
from jax.experimental.pallas import tpu_sc as plsc

NSEG = 512
DIM = 256
NROWS = 160000
NC, NS, LANES = 2, 16, 16
NW = NC * NS

TCBLK = 1600
NTCB = 28
TCROWS = TCBLK * NTCB
WROWS = (NROWS - TCROWS) // NW
CHUNK = 160
NCH = (WROWS + CHUNK - 1) // CHUNK
NCHP = NCH + (NCH % 2)
NT = DIM // LANES
SIDE_W = DIM + 2 * LANES
CNTW = 16
ZROWS = NSEG // NS


def _sc_segmean(feat, idx1, sum_core, cnt_core, side,
                fbuf_a, fbuf_b, ibuf_a, ibuf_b, stg, stgc, zb, zcb,
                sem_a, sem_b):
    cid = lax.axis_index("c")
    sid = lax.axis_index("s")
    wid = sid * NC + cid

    zeros = jnp.zeros((LANES,), jnp.float32)
    f32 = jnp.float32

    def splat_f(x):
        return zeros + x.astype(f32)

    def zero_zrow(r, carry):
        for j in range(NT):
            zb[r, pl.ds(j * LANES, LANES)] = zeros
        zcb[r, pl.ds(0, LANES)] = zeros
        return carry
    lax.fori_loop(0, ZROWS, zero_zrow, None)
    for j in range(SIDE_W // LANES):
        stg[0, pl.ds(j * LANES, LANES)] = zeros
    stgc[0, pl.ds(0, LANES)] = zeros

    rows16 = pl.ds(sid * ZROWS, ZROWS)
    pltpu.sync_copy(zb, sum_core.at[cid].at[rows16])
    pltpu.sync_copy(zcb, cnt_core.at[cid].at[rows16])
    plsc.subcore_barrier()

    r0 = TCROWS + wid * WROWS

    def fill_stg(cur, n, acc):
        for t in range(NT):
            stg[0, pl.ds(t * LANES, LANES)] = acc[t]
        stg[0, pl.ds(DIM, LANES)] = splat_f(n)
        stg[0, pl.ds(DIM + LANES, LANES)] = splat_f(cur)

    def flush(cur, n, ff, acc):
        fill_stg(cur, n, acc)

        @pl.when(ff == 0)
        def _():
            pltpu.sync_copy(stg, side.at[pl.ds(2 * wid, 1)])

        @pl.when(ff != 0)
        def _():
            pltpu.sync_copy(stg.at[:, pl.ds(0, DIM)],
                            sum_core.at[cid].at[pl.ds(cur, 1)])
            stgc[0, pl.ds(0, LANES)] = splat_f(n)
            pltpu.sync_copy(stgc, cnt_core.at[cid].at[pl.ds(cur, 1)])

    def chunk_base(k):
        gbase = r0 + k * CHUNK
        return gbase, jnp.minimum(gbase, NROWS - CHUNK)

    def start_dma(k, fb, ib, sem):
        _, cbase = chunk_base(k)
        pltpu.async_copy(feat.at[pl.ds(cbase, CHUNK)], fb, sem)
        pltpu.async_copy(idx1.at[pl.ds(cbase, CHUNK)], ib.at[pl.ds(0, CHUNK)],
                         sem)

    def wait_dma(k, fb, ib, sem):
        _, cbase = chunk_base(k)
        pltpu.make_async_copy(feat.at[pl.ds(cbase, CHUNK)], fb, sem).wait()
        pltpu.make_async_copy(idx1.at[pl.ds(cbase, CHUNK)],
                              ib.at[pl.ds(0, CHUNK)], sem).wait()

    izeros = jnp.zeros((LANES,), jnp.int32)

    def process(k, fbuf, ibuf, st):
        gbase, cbase = chunk_base(k)
        off = gbase - cbase
        hi = off + jnp.minimum(CHUNK, WROWS - k * CHUNK)

        def iread(j):
            return jnp.max(plsc.load_gather(ibuf, [izeros + j]))

        def row_add(j, acc):
            return tuple(acc[t] + fbuf[j, pl.ds(t * LANES, LANES)]
                         for t in range(NT))

        def row_slow(j, st):
            cur, n, ff = st[0], st[1], st[2]
            acc = st[3:]
            s = iread(j)
            changed = s != cur

            @pl.when(jnp.logical_and(changed, cur >= 0))
            def _():
                flush(cur, n, ff, acc)

            newseg = jnp.logical_and(changed, cur >= 0)
            ff = jnp.where(newseg, 1, ff)
            n = jnp.where(changed, 0, n) + 1
            acc = tuple(
                jnp.where(changed, zeros, acc[t])
                + fbuf[j, pl.ds(t * LANES, LANES)]
                for t in range(NT)
            )
            return (s, n, ff) + acc

        zero32 = jnp.int32(0)

        def group_step(g, st):
            cur, n, ff = st[0], st[1], st[2]
            j0 = off + LANES * g
            ghi = jnp.minimum(j0 + LANES, hi)
            iv = ibuf[pl.ds(j0, LANES)]
            gfast = jnp.all(iv == izeros + cur)
            acc = lax.fori_loop(jnp.where(gfast, j0, zero32),
                                jnp.where(gfast, ghi, zero32),
                                row_add, st[3:])
            n = n + jnp.where(gfast, ghi - j0, 0)
            return lax.fori_loop(jnp.where(gfast, zero32, j0),
                                 jnp.where(gfast, zero32, ghi),
                                 row_slow, (cur, n, ff) + acc)

        ngroups = (hi - off + LANES - 1) // LANES
        return lax.fori_loop(0, ngroups, group_step, st)

    init = (jnp.int32(-1), jnp.int32(0), jnp.int32(0)) + tuple(
        zeros for _ in range(NT))

    start_dma(jnp.int32(0), fbuf_a, ibuf_a, sem_a)

    def pair_step(k2, st):
        k = 2 * k2
        wait_dma(k, fbuf_a, ibuf_a, sem_a)
        start_dma(k + 1, fbuf_b, ibuf_b, sem_b)
        st = process(k, fbuf_a, ibuf_a, st)
        wait_dma(k + 1, fbuf_b, ibuf_b, sem_b)

        @pl.when(k + 2 < NCHP)
        def _():
            start_dma(k + 2, fbuf_a, ibuf_a, sem_a)
        return process(k + 1, fbuf_b, ibuf_b, st)

    st = lax.fori_loop(0, NCHP // 2, pair_step, init)
    cur, n, ff = st[0], st[1], st[2]
    acc = st[3:]

    fill_stg(cur, n, acc)
    pltpu.sync_copy(stg, side.at[pl.ds(2 * wid + 1, 1)])

    @pl.when(ff == 0)
    def _():
        stg[0, pl.ds(DIM, LANES)] = zeros
        stg[0, pl.ds(DIM + LANES, LANES)] = splat_f(jnp.int32(-1))
        pltpu.sync_copy(stg, side.at[pl.ds(2 * wid, 1)])


def _tc_partials(idx_ref, feat_ref, sum_ref, cnt_ref):
    i = pl.program_id(0)
    ids = idx_ref[0]
    seg = lax.broadcasted_iota(jnp.int32, (NSEG, TCBLK), 0)
    oh = jnp.where(ids == seg, 1.0, 0.0).astype(jnp.bfloat16)
    fb = feat_ref[...].astype(jnp.bfloat16)
    part = jnp.dot(oh, fb, preferred_element_type=jnp.float32)
    ones_c = jnp.ones((TCBLK, 1), jnp.bfloat16)
    cnt = jnp.dot(oh, ones_c, preferred_element_type=jnp.float32)

    @pl.when(i == 0)
    def _():
        sum_ref[...] = part
        cnt_ref[...] = cnt

    @pl.when(i > 0)
    def _():
        sum_ref[...] = sum_ref[...] + part
        cnt_ref[...] = cnt_ref[...] + cnt


def _combine_body(sum_ref, cnt_ref, side_ref, tsum_ref, tcnt_ref, out_ref):
    sums = sum_ref[0] + sum_ref[1] + tsum_ref[...]
    cnts = cnt_ref[0][:, 0:1] + cnt_ref[1][:, 0:1] + tcnt_ref[...]
    side = side_ref[...]
    ids = side[:, DIM + LANES:DIM + LANES + 1]
    seg_iota = lax.broadcasted_iota(jnp.int32, (NW * 2, NSEG), 1).astype(
        jnp.float32)
    onehot = (ids == seg_iota).astype(jnp.float32)
    side_c = lax.dot_general(onehot, side,
                             (((0,), (0,)), ((), ())),
                             preferred_element_type=jnp.float32)
    total = sums + side_c[:, :DIM]
    cnt = cnts + side_c[:, DIM:DIM + 1]
    out_ref[...] = total / jnp.maximum(cnt, 1.0)


def kernel(node_features, batch_index):
    idx1 = batch_index.astype(jnp.int32)

    sc = pl.kernel(
        _sc_segmean,
        out_type=[
            jax.ShapeDtypeStruct((NC, NSEG, DIM), jnp.float32),
            jax.ShapeDtypeStruct((NC, NSEG, CNTW), jnp.float32),
            jax.ShapeDtypeStruct((NW * 2, SIDE_W), jnp.float32),
        ],
        mesh=plsc.VectorSubcoreMesh(
            core_axis_name="c", subcore_axis_name="s",
            num_cores=NC, num_subcores=NS,
        ),
        compiler_params=pltpu.CompilerParams(needs_layout_passes=False),
        scratch_types=[
            pltpu.VMEM((CHUNK, DIM), jnp.float32),
            pltpu.VMEM((CHUNK, DIM), jnp.float32),
            pltpu.VMEM((CHUNK + LANES,), jnp.int32),
            pltpu.VMEM((CHUNK + LANES,), jnp.int32),
            pltpu.VMEM((1, SIDE_W), jnp.float32),
            pltpu.VMEM((1, CNTW), jnp.float32),
            pltpu.VMEM((ZROWS, DIM), jnp.float32),
            pltpu.VMEM((ZROWS, CNTW), jnp.float32),
            pltpu.SemaphoreType.DMA,
            pltpu.SemaphoreType.DMA,
        ],
    )
    sum_core, cnt_core, side = sc(node_features, idx1)

    idx3 = idx1.reshape(NROWS // TCBLK, 1, TCBLK)
    tsum, tcnt = pl.pallas_call(
        _tc_partials,
        grid=(NTCB,),
        in_specs=[
            pl.BlockSpec((1, 1, TCBLK), lambda i: (i, 0, 0)),
            pl.BlockSpec((TCBLK, DIM), lambda i: (i, 0)),
        ],
        out_specs=[
            pl.BlockSpec((NSEG, DIM), lambda i: (0, 0)),
            pl.BlockSpec((NSEG, 1), lambda i: (0, 0)),
        ],
        out_shape=[
            jax.ShapeDtypeStruct((NSEG, DIM), jnp.float32),
            jax.ShapeDtypeStruct((NSEG, 1), jnp.float32),
        ],
    )(idx3, node_features)

    out = pl.pallas_call(
        _combine_body,
        out_shape=jax.ShapeDtypeStruct((NSEG, DIM), jnp.float32),
    )(sum_core, cnt_core, side, tsum, tcnt)
    return out
